# baseline (speedup 1.0000x reference)
import functools

import jax
import jax.numpy as jnp
from jax.experimental import pallas as pl
from jax.experimental.pallas import tpu as pltpu

_N = 7
_E = 14
_F0 = 16
_F1 = 8
_D0 = _N * _F0
_D1 = _N * _F1

_dot = functools.partial(
    jax.lax.dot_general, preferred_element_type=jnp.float32)
_C11 = (((1,), (1,)), ((), ()))
_C10 = (((1,), (0,)), ((), ()))


def _eq_iota(shape, div, mod=None):
    r = jax.lax.broadcasted_iota(jnp.int32, shape, 0)
    c = jax.lax.broadcasted_iota(jnp.int32, shape, 1)
    if mod is None:
        return (r // div == c).astype(jnp.float32)
    return (r % mod == c).astype(jnp.float32)


def _prep(ei_ref, w1_ref, w2_ref, b1_ref, b2_ref,
          m1_scr, m2_scr, b1t_scr, b2t_scr):
    f32 = jnp.float32
    src = ei_ref[0:1, :]
    dst = ei_ref[1:2, :]
    rows = jax.lax.broadcasted_iota(jnp.int32, (_N, _E), 0)
    oh_src = (rows == src).astype(f32)
    oh_dst = (rows == dst).astype(f32)

    deg = jnp.sum(oh_dst, axis=1, keepdims=True) + 1.0
    dinv = jax.lax.rsqrt(deg)

    dinv_src = jnp.sum(dinv * oh_src, axis=0, keepdims=True)
    dinv_dst = jnp.sum(dinv * oh_dst, axis=0, keepdims=True)
    norm = dinv_src * dinv_dst

    A = _dot(oh_dst * norm, oh_src, _C11)
    eye_r = jax.lax.broadcasted_iota(jnp.int32, (_N, _N), 0)
    eye_c = jax.lax.broadcasted_iota(jnp.int32, (_N, _N), 1)
    A = A + (eye_r == eye_c).astype(f32) * (dinv * dinv)

    R16 = _eq_iota((_D0, _N), _F0)
    C8 = _eq_iota((_D1, _N), _F1)
    T16 = _eq_iota((_D0, _F0), 1, _F0)
    T8 = _eq_iota((_D1, _F1), 1, _F1)

    repA1 = _dot(_dot(R16, A, _C11), C8, _C11)
    tileW1 = _dot(_dot(T16, w1_ref[...], _C10), T8, _C11)
    m1_scr[...] = repA1 * tileW1

    repA2 = _dot(_dot(C8, A, _C11), C8, _C11)
    tileW2 = _dot(_dot(T8, w2_ref[...], _C10), T8, _C11)
    m2_scr[...] = repA2 * tileW2

    b1t_scr[...] = _dot(b1_ref[...], T8, _C11)
    b2t_scr[...] = _dot(b2_ref[...], T8, _C11)


_K = 4
_T = 4096


def _body(ei_ref, w1_ref, w2_ref, b1_ref, b2_ref,
          wl1_ref, wl2_ref, bl1_ref, bl2_ref, *refs):
    x_refs = refs[:_K]
    o_refs = refs[_K:2 * _K]
    m1_scr, m2_scr, b1t_scr, b2t_scr = refs[2 * _K:]

    @pl.when(pl.program_id(0) == 0)
    def _():
        _prep(ei_ref, w1_ref, w2_ref, b1_ref, b2_ref,
              m1_scr, m2_scr, b1t_scr, b2t_scr)

    bf16 = jnp.bfloat16
    m1b = m1_scr[...].astype(bf16)
    m2b = m2_scr[...].astype(bf16)
    wl1b = wl1_ref[...].astype(bf16)
    wl2b = wl2_ref[...].astype(bf16)
    for j in range(_K):
        xb = x_refs[j][...].astype(bf16)
        t = jnp.maximum(_dot(xb, m1b, _C10) + b1t_scr[...], 0.0)
        t = jnp.maximum(_dot(t.astype(bf16), m2b, _C10) + b2t_scr[...], 0.0)
        t = jnp.maximum(_dot(t.astype(bf16), wl1b, _C10) + bl1_ref[...], 0.0)
        o_refs[j][...] = _dot(t.astype(bf16), wl2b, _C10) + bl2_ref[...]


def kernel(x1, edge_index, W1, b1, W2, b2, Wl1, bl1, Wl2, bl2):
    B = x1.shape[0]
    S = B // (_K * _T)
    rep = lambda i: (0, 0)

    def xmap(j):
        return lambda i: (i * _K + j, 0)

    outs = pl.pallas_call(
        _body,
        grid=(S,),
        in_specs=[
            pl.BlockSpec((2, _E), rep),
            pl.BlockSpec((_F0, _F1), rep),
            pl.BlockSpec((_F1, _F1), rep),
            pl.BlockSpec((1, _F1), rep),
            pl.BlockSpec((1, _F1), rep),
            pl.BlockSpec((_D1, 24), rep),
            pl.BlockSpec((24, 1), rep),
            pl.BlockSpec((1, 24), rep),
            pl.BlockSpec((1, 1), rep),
        ] + [pl.BlockSpec((_T, _D0), xmap(j)) for j in range(_K)],
        out_specs=[pl.BlockSpec((_T, 1), lambda i: (i, 0))] * _K,
        out_shape=[jax.ShapeDtypeStruct((S * _T, 1), jnp.float32)] * _K,
        scratch_shapes=[
            pltpu.VMEM((_D0, _D1), jnp.float32),
            pltpu.VMEM((_D1, _D1), jnp.float32),
            pltpu.VMEM((1, _D1), jnp.float32),
            pltpu.VMEM((1, _D1), jnp.float32),
        ],
    )(edge_index, W1, W2, b1.reshape(1, -1), b2.reshape(1, -1),
      Wl1, Wl2, bl1.reshape(1, -1), bl2.reshape(1, -1),
      *[x1 for _ in range(_K)])
    if _K == 1:
        return outs[0]
    out = jnp.stack([o.reshape(S, _T) for o in outs], axis=1)
    return out.reshape(B, 1)


# speedup vs baseline: 1.0334x; 1.0334x over previous
import functools

import jax
import jax.numpy as jnp
from jax.experimental import pallas as pl
from jax.experimental.pallas import tpu as pltpu

_N = 7
_E = 14
_F0 = 16
_F1 = 8
_D0 = _N * _F0
_D1 = _N * _F1

_dot = functools.partial(
    jax.lax.dot_general, preferred_element_type=jnp.float32)
_C11 = (((1,), (1,)), ((), ()))
_C10 = (((1,), (0,)), ((), ()))


def _eq_iota(shape, div, mod=None):
    r = jax.lax.broadcasted_iota(jnp.int32, shape, 0)
    c = jax.lax.broadcasted_iota(jnp.int32, shape, 1)
    if mod is None:
        return (r // div == c).astype(jnp.float32)
    return (r % mod == c).astype(jnp.float32)


def _prep(ei_ref, w1_ref, w2_ref, b1_ref, b2_ref,
          m1_scr, m2_scr, b1t_scr, b2t_scr):
    f32 = jnp.float32
    src = ei_ref[0:1, :]
    dst = ei_ref[1:2, :]
    rows = jax.lax.broadcasted_iota(jnp.int32, (_N, _E), 0)
    oh_src = (rows == src).astype(f32)
    oh_dst = (rows == dst).astype(f32)

    deg = jnp.sum(oh_dst, axis=1, keepdims=True) + 1.0
    dinv = jax.lax.rsqrt(deg)

    dinv_src = jnp.sum(dinv * oh_src, axis=0, keepdims=True)
    dinv_dst = jnp.sum(dinv * oh_dst, axis=0, keepdims=True)
    norm = dinv_src * dinv_dst

    A = _dot(oh_dst * norm, oh_src, _C11)
    eye_r = jax.lax.broadcasted_iota(jnp.int32, (_N, _N), 0)
    eye_c = jax.lax.broadcasted_iota(jnp.int32, (_N, _N), 1)
    A = A + (eye_r == eye_c).astype(f32) * (dinv * dinv)

    R16 = _eq_iota((_D0, _N), _F0)
    C8 = _eq_iota((_D1, _N), _F1)
    T16 = _eq_iota((_D0, _F0), 1, _F0)
    T8 = _eq_iota((_D1, _F1), 1, _F1)

    repA1 = _dot(_dot(R16, A, _C11), C8, _C11)
    tileW1 = _dot(_dot(T16, w1_ref[...], _C10), T8, _C11)
    m1_scr[...] = repA1 * tileW1

    repA2 = _dot(_dot(C8, A, _C11), C8, _C11)
    tileW2 = _dot(_dot(T8, w2_ref[...], _C10), T8, _C11)
    m2_scr[...] = repA2 * tileW2

    b1t_scr[...] = _dot(b1_ref[...], T8, _C11)
    b2t_scr[...] = _dot(b2_ref[...], T8, _C11)


_K = 2
_T = 8192


def _body(ei_ref, w1_ref, w2_ref, b1_ref, b2_ref,
          wl1_ref, wl2_ref, bl1_ref, bl2_ref, *refs):
    x_refs = refs[:_K]
    o_refs = refs[_K:2 * _K]
    m1_scr, m2_scr, b1t_scr, b2t_scr = refs[2 * _K:]

    @pl.when(pl.program_id(0) == 0)
    def _():
        _prep(ei_ref, w1_ref, w2_ref, b1_ref, b2_ref,
              m1_scr, m2_scr, b1t_scr, b2t_scr)

    for j in range(_K):
        t = jnp.maximum(
            _dot(x_refs[j][...], m1_scr[...], _C10) + b1t_scr[...], 0.0)
        t = jnp.maximum(_dot(t, m2_scr[...], _C10) + b2t_scr[...], 0.0)
        t = jnp.maximum(_dot(t, wl1_ref[...], _C10) + bl1_ref[...], 0.0)
        o_refs[j][...] = _dot(t, wl2_ref[...], _C10) + bl2_ref[...]


def kernel(x1, edge_index, W1, b1, W2, b2, Wl1, bl1, Wl2, bl2):
    B = x1.shape[0]
    S = B // (_K * _T)
    rep = lambda i: (0, 0)

    def xmap(j):
        return lambda i: (i * _K + j, 0)

    outs = pl.pallas_call(
        _body,
        grid=(S,),
        in_specs=[
            pl.BlockSpec((2, _E), rep),
            pl.BlockSpec((_F0, _F1), rep),
            pl.BlockSpec((_F1, _F1), rep),
            pl.BlockSpec((1, _F1), rep),
            pl.BlockSpec((1, _F1), rep),
            pl.BlockSpec((_D1, 24), rep),
            pl.BlockSpec((24, 1), rep),
            pl.BlockSpec((1, 24), rep),
            pl.BlockSpec((1, 1), rep),
        ] + [pl.BlockSpec((_T, _D0), xmap(j)) for j in range(_K)],
        out_specs=[pl.BlockSpec((_T, 1), lambda i: (i, 0))] * _K,
        out_shape=[jax.ShapeDtypeStruct((S * _T, 1), jnp.float32)] * _K,
        scratch_shapes=[
            pltpu.VMEM((_D0, _D1), jnp.float32),
            pltpu.VMEM((_D1, _D1), jnp.float32),
            pltpu.VMEM((1, _D1), jnp.float32),
            pltpu.VMEM((1, _D1), jnp.float32),
        ],
    )(edge_index, W1, W2, b1.reshape(1, -1), b2.reshape(1, -1),
      Wl1, Wl2, bl1.reshape(1, -1), bl2.reshape(1, -1),
      *[x1 for _ in range(_K)])
    if _K == 1:
        return outs[0]
    out = jnp.stack([o.reshape(S, _T) for o in outs], axis=1)
    return out.reshape(B, 1)
